# baseline (speedup 1.0000x reference)
import functools

import jax
import jax.numpy as jnp
from jax import lax
from jax.experimental import pallas as pl
from jax.experimental.pallas import tpu as pltpu
from jax.experimental.pallas import tpu_sc as plsc

_NC = 2
_NS = 16
_NW = _NC * _NS
_B = 4096
_E = 16
_BPW = _B // _NW
_NF = 9
_PAIRS = [(i, j) for i in range(_NF) for j in range(i + 1, _NF)]
_NP = len(_PAIRS)

_SMALL_PAD = {"cluster": 1024, "sex": 128, "country": 256,
              "province": 128, "city": 512}
_SMALL_ORDER = ["cluster", "sex", "country", "province", "city"]
_SMALL_OFF = {"cluster": 0, "sex": 1024, "country": 1152,
              "province": 1408, "city": 1536}
_SMALL_TOTAL = 2048


def _iota16():
    return lax.iota(jnp.int32, 16)


_NG = _BPW // 16


def _sc_gather_body(*refs):
    idxs = refs[0:7]
    big = refs[7:9]
    small = refs[9]
    out = refs[10]
    vidx = refs[11:18]
    gbuf = refs[18:20]
    sbuf = refs[20]
    obuf = refs[21:28]
    sem_big = refs[28]
    sem_pre = refs[29]
    sem_out = refs[30]

    wid = lax.axis_index("s") * _NC + lax.axis_index("c")
    base = wid * _BPW
    i16 = _iota16()

    idx_dma = [pltpu.async_copy(idxs[t].at[pl.ds(base, _BPW)], vidx[t], sem_pre)
               for t in range(7)]
    pre = pltpu.async_copy(small, sbuf, sem_pre)

    def fire_group(tab, ivec, buf):
        dmas = []
        for k in range(16):
            off = pl.multiple_of(jnp.bitwise_and(ivec[k], ~127), 128)
            dmas.append(pltpu.async_copy(
                tab.at[:, pl.ds(off, 128)], buf.at[:, pl.ds(128 * k, 128)],
                sem_big))
        return dmas

    def extract_group(ivec, buf, ob, g):
        lanes = 128 * i16 + jnp.bitwise_and(ivec, 127)
        for j in range(16):
            row = plsc.load_gather(buf, [jnp.full((16,), j, jnp.int32), lanes])
            ob[j, pl.ds(16 * g, 16)] = row

    for bt in range(2):
        tab = big[bt]
        idx_dma[bt].wait()
        ivecs = [vidx[bt][pl.ds(16 * g, 16)] for g in range(_NG)]
        dmas = [fire_group(tab, ivecs[0], gbuf[0])]
        for g in range(_NG):
            if g + 1 < _NG:
                dmas.append(fire_group(tab, ivecs[g + 1], gbuf[(g + 1) % 2]))
            for d in dmas[g]:
                d.wait()
            extract_group(ivecs[g], gbuf[g % 2], obuf[bt], g)
        pltpu.async_copy(obuf[bt],
                         out.at[pl.ds(16 * bt, 16), pl.ds(base, _BPW)],
                         sem_out)

    for t in range(2, 7):
        idx_dma[t].wait()
    pre.wait()

    for st, name in enumerate(_SMALL_ORDER):
        off = _SMALL_OFF[name]

        def small_group(k0, _, st=st, off=off):
            ivec = off + vidx[2 + st][pl.ds(k0 * 16, 16)]
            for j in range(16):
                row = plsc.load_gather(sbuf, [jnp.full((16,), j, jnp.int32), ivec])
                obuf[2 + st][j, pl.ds(k0 * 16, 16)] = row
            return 0

        lax.fori_loop(0, _NG, small_group, 0)

    odma = [pltpu.async_copy(obuf[t],
                             out.at[pl.ds(16 * t, 16), pl.ds(base, _BPW)],
                             sem_out)
            for t in range(2, 7)]
    for _ in range(2):
        pltpu.make_async_copy(
            obuf[0], out.at[pl.ds(0, 16), pl.ds(base, _BPW)], sem_out).wait()
    for d in odma:
        d.wait()


@functools.cache
def _sc_gather():
    mesh = plsc.VectorSubcoreMesh(core_axis_name="c", subcore_axis_name="s")
    return functools.partial(
        pl.kernel,
        mesh=mesh,
        compiler_params=pltpu.CompilerParams(needs_layout_passes=False),
        out_type=jax.ShapeDtypeStruct((112, _B), jnp.float32),
        scratch_types=(
            [pltpu.VMEM((_BPW,), jnp.int32)] * 7
            + [pltpu.VMEM((16, 2048), jnp.float32)] * 2
            + [pltpu.VMEM((16, _SMALL_TOTAL), jnp.float32)]
            + [pltpu.VMEM((16, _BPW), jnp.float32)] * 7
            + [pltpu.SemaphoreType.DMA] * 3
        ),
    )(_sc_gather_body)


_BB = 4096


def _mmT(lhs, rhs):
    return lax.dot_general(lhs, rhs, (((0,), (0,)), ((), ())),
                           preferred_element_type=jnp.float32)


def _tc_body(gath, dense, video,
             bw1, bb1, bw2, bb2, vpw, vpb,
             tw1, tb1, tw2, tb2, tw3, tb3, out):
    f32 = jnp.float32
    h = jnp.maximum(
        lax.dot_general(bw1[...], dense[...], (((0,), (1,)), ((), ())),
                        preferred_element_type=f32) + bb1[...], 0.0)
    v_dense = jnp.maximum(_mmT(bw2[...], h) + bb2[...], 0.0)
    v_video = lax.dot_general(vpw[...], video[...], (((0,), (1,)), ((), ())),
                              preferred_element_type=f32) + vpb[...]
    g = gath[...]
    fields = [g[16 * t:16 * t + 16] for t in range(7)] + [v_dense, v_video]
    l = jnp.concatenate([fields[i] for i, _ in _PAIRS], axis=0)
    r = jnp.concatenate([fields[j] for _, j in _PAIRS], axis=0)
    p = l * r
    sel = (lax.broadcasted_iota(jnp.int32, (_NP * _E, _NP), 0) // _E
           == lax.broadcasted_iota(jnp.int32, (_NP * _E, _NP), 1)).astype(f32)
    inter = _mmT(sel, p)
    x = jnp.concatenate([inter, v_dense], axis=0)
    h2 = jnp.maximum(_mmT(tw1[...], x) + tb1[...], 0.0)
    h3 = jnp.maximum(_mmT(tw2[...], h2) + tb2[...], 0.0)
    logit = _mmT(tw3[...], h3) + tb3[...]
    out[...] = 1.0 / (1.0 + jnp.exp(-logit))


def kernel(user, item, cluster, sex, country, province, city,
           emb_user, emb_item, emb_cluster, emb_sex, emb_country,
           emb_province, emb_city,
           dense_inputs, video_vector,
           bm_w1, bm_b1, bm_w2, bm_b2, vp_w, vp_b,
           tm_w1, tm_b1, tm_w2, tm_b2, tm_w3, tm_b3):
    smalls = {"cluster": emb_cluster, "sex": emb_sex, "country": emb_country,
              "province": emb_province, "city": emb_city}
    small_cat = jnp.concatenate(
        [jnp.pad(smalls[n].T,
                 ((0, 0), (0, _SMALL_PAD[n] - smalls[n].shape[0])))
         for n in _SMALL_ORDER], axis=1)
    gathered = _sc_gather()(
        user, item, cluster, sex, country, province, city,
        emb_user.T, emb_item.T, small_cat)

    grid = _B // _BB
    in_specs = [
        pl.BlockSpec((112, _BB), lambda i: (0, i)),
        pl.BlockSpec((_BB, 2), lambda i: (i, 0)),
        pl.BlockSpec((_BB, 512), lambda i: (i, 0)),
    ] + [pl.BlockSpec(s, lambda i: (0, 0)) for s in
         [(2, 32), (32, 1), (32, 16), (16, 1), (512, 16), (16, 1),
          (52, 64), (64, 1), (64, 32), (32, 1), (32, 1), (1, 1)]]
    out_t = pl.pallas_call(
        _tc_body,
        grid=(grid,),
        in_specs=in_specs,
        out_specs=pl.BlockSpec((1, _BB), lambda i: (0, i)),
        out_shape=jax.ShapeDtypeStruct((1, _B), jnp.float32),
    )(gathered, dense_inputs, video_vector,
      bm_w1, bm_b1.reshape(32, 1), bm_w2, bm_b2.reshape(16, 1),
      vp_w, vp_b.reshape(16, 1),
      tm_w1, tm_b1.reshape(64, 1), tm_w2, tm_b2.reshape(32, 1),
      tm_w3, tm_b3.reshape(1, 1))
    return out_t.reshape(_B, 1)


# speedup vs baseline: 1.0173x; 1.0173x over previous
import functools

import jax
import jax.numpy as jnp
from jax import lax
from jax.experimental import pallas as pl
from jax.experimental.pallas import tpu as pltpu
from jax.experimental.pallas import tpu_sc as plsc

_NC = 2
_NS = 16
_NW = _NC * _NS
_B = 4096
_E = 16
_BPW = _B // _NW
_NF = 9
_PAIRS = [(i, j) for i in range(_NF) for j in range(i + 1, _NF)]
_NP = len(_PAIRS)

_SMALL_PAD = {"cluster": 1024, "sex": 128, "country": 256,
              "province": 128, "city": 512}
_SMALL_ORDER = ["cluster", "sex", "country", "province", "city"]
_SMALL_OFF = {"cluster": 0, "sex": 1024, "country": 1152,
              "province": 1408, "city": 1536}
_SMALL_TOTAL = 2048


def _iota16():
    return lax.iota(jnp.int32, 16)


_NG = _BPW // 16


def _sc_gather_body(*refs):
    idxs = refs[0:7]
    big = refs[7:9]
    small = refs[9]
    out = refs[10]
    vidx = refs[11:18]
    gbuf = refs[18:20]
    sbuf = refs[20]
    obuf = refs[21:28]
    sem_big = refs[28]
    sem_pre = refs[29]
    sem_out = refs[30]

    wid = lax.axis_index("s") * _NC + lax.axis_index("c")
    base = wid * _BPW
    i16 = _iota16()

    idx_dma = [pltpu.async_copy(idxs[t].at[pl.ds(base, _BPW)], vidx[t], sem_pre)
               for t in range(7)]
    pre = pltpu.async_copy(small, sbuf, sem_pre)

    def fire_group(tab, ivec, buf):
        dmas = []
        for k in range(16):
            off = pl.multiple_of(jnp.bitwise_and(ivec[k], ~127), 128)
            dmas.append(pltpu.async_copy(
                tab.at[:, pl.ds(off, 128)], buf.at[:, pl.ds(128 * k, 128)],
                sem_big))
        return dmas

    def extract_group(ivec, buf, ob, g):
        lanes = 128 * i16 + jnp.bitwise_and(ivec, 127)
        for j in range(16):
            row = plsc.load_gather(buf, [jnp.full((16,), j, jnp.int32), lanes])
            ob[j, pl.ds(16 * g, 16)] = row

    for bt in range(2):
        tab = big[bt]
        idx_dma[bt].wait()
        ivecs = [vidx[bt][pl.ds(16 * g, 16)] for g in range(_NG)]
        dmas = [fire_group(tab, ivecs[0], gbuf[0])]
        for g in range(_NG):
            if g + 1 < _NG:
                dmas.append(fire_group(tab, ivecs[g + 1], gbuf[(g + 1) % 2]))
            for d in dmas[g]:
                d.wait()
            extract_group(ivecs[g], gbuf[g % 2], obuf[bt], g)
        pltpu.async_copy(obuf[bt],
                         out.at[pl.ds(16 * bt, 16), pl.ds(base, _BPW)],
                         sem_out)

    for t in range(2, 7):
        idx_dma[t].wait()
    pre.wait()

    for st, name in enumerate(_SMALL_ORDER):
        off = _SMALL_OFF[name]

        def small_group(k0, _, st=st, off=off):
            ivec = off + vidx[2 + st][pl.ds(k0 * 16, 16)]
            for j in range(16):
                row = plsc.load_gather(sbuf, [jnp.full((16,), j, jnp.int32), ivec])
                obuf[2 + st][j, pl.ds(k0 * 16, 16)] = row
            return 0

        lax.fori_loop(0, _NG, small_group, 0)

    odma = [pltpu.async_copy(obuf[t],
                             out.at[pl.ds(16 * t, 16), pl.ds(base, _BPW)],
                             sem_out)
            for t in range(2, 7)]
    for _ in range(2):
        pltpu.make_async_copy(
            obuf[0], out.at[pl.ds(0, 16), pl.ds(base, _BPW)], sem_out).wait()
    for d in odma:
        d.wait()


@functools.cache
def _sc_gather():
    mesh = plsc.VectorSubcoreMesh(core_axis_name="c", subcore_axis_name="s")
    return functools.partial(
        pl.kernel,
        mesh=mesh,
        compiler_params=pltpu.CompilerParams(needs_layout_passes=False),
        out_type=jax.ShapeDtypeStruct((112, _B), jnp.float32),
        scratch_types=(
            [pltpu.VMEM((_BPW,), jnp.int32)] * 7
            + [pltpu.VMEM((16, 2048), jnp.float32)] * 2
            + [pltpu.VMEM((16, _SMALL_TOTAL), jnp.float32)]
            + [pltpu.VMEM((16, _BPW), jnp.float32)] * 7
            + [pltpu.SemaphoreType.DMA] * 3
        ),
    )(_sc_gather_body)


_BB = 2048


def _mmT(lhs, rhs):
    return lax.dot_general(lhs, rhs, (((0,), (0,)), ((), ())),
                           preferred_element_type=jnp.float32)


def _tc_body(gath, dense, video,
             bw1, bb1, bw2, bb2, vpw, vpb,
             tw1, tb1, tw2, tb2, tw3, tb3, out):
    f32 = jnp.float32
    h = jnp.maximum(
        lax.dot_general(bw1[...], dense[...], (((0,), (1,)), ((), ())),
                        preferred_element_type=f32) + bb1[...], 0.0)
    v_dense = jnp.maximum(_mmT(bw2[...], h) + bb2[...], 0.0)
    v_video = lax.dot_general(vpw[...], video[...], (((0,), (1,)), ((), ())),
                              preferred_element_type=f32) + vpb[...]
    g = gath[...]
    fields = [g[16 * t:16 * t + 16] for t in range(7)] + [v_dense, v_video]
    l = jnp.concatenate([fields[i] for i, _ in _PAIRS], axis=0)
    r = jnp.concatenate([fields[j] for _, j in _PAIRS], axis=0)
    p = l * r
    sel = (lax.broadcasted_iota(jnp.int32, (_NP * _E, _NP), 0) // _E
           == lax.broadcasted_iota(jnp.int32, (_NP * _E, _NP), 1)).astype(f32)
    inter = _mmT(sel, p)
    x = jnp.concatenate([inter, v_dense], axis=0)
    h2 = jnp.maximum(_mmT(tw1[...], x) + tb1[...], 0.0)
    h3 = jnp.maximum(_mmT(tw2[...], h2) + tb2[...], 0.0)
    logit = _mmT(tw3[...], h3) + tb3[...]
    out[...] = 1.0 / (1.0 + jnp.exp(-logit))


def kernel(user, item, cluster, sex, country, province, city,
           emb_user, emb_item, emb_cluster, emb_sex, emb_country,
           emb_province, emb_city,
           dense_inputs, video_vector,
           bm_w1, bm_b1, bm_w2, bm_b2, vp_w, vp_b,
           tm_w1, tm_b1, tm_w2, tm_b2, tm_w3, tm_b3):
    smalls = {"cluster": emb_cluster, "sex": emb_sex, "country": emb_country,
              "province": emb_province, "city": emb_city}
    small_cat = jnp.concatenate(
        [jnp.pad(smalls[n].T,
                 ((0, 0), (0, _SMALL_PAD[n] - smalls[n].shape[0])))
         for n in _SMALL_ORDER], axis=1)
    gathered = _sc_gather()(
        user, item, cluster, sex, country, province, city,
        emb_user.T, emb_item.T, small_cat)

    grid = _B // _BB
    in_specs = [
        pl.BlockSpec((112, _BB), lambda i: (0, i)),
        pl.BlockSpec((_BB, 2), lambda i: (i, 0)),
        pl.BlockSpec((_BB, 512), lambda i: (i, 0)),
    ] + [pl.BlockSpec(s, lambda i: (0, 0)) for s in
         [(2, 32), (32, 1), (32, 16), (16, 1), (512, 16), (16, 1),
          (52, 64), (64, 1), (64, 32), (32, 1), (32, 1), (1, 1)]]
    out_t = pl.pallas_call(
        _tc_body,
        grid=(grid,),
        in_specs=in_specs,
        out_specs=pl.BlockSpec((1, _BB), lambda i: (0, i)),
        out_shape=jax.ShapeDtypeStruct((1, _B), jnp.float32),
    )(gathered, dense_inputs, video_vector,
      bm_w1, bm_b1.reshape(32, 1), bm_w2, bm_b2.reshape(16, 1),
      vp_w, vp_b.reshape(16, 1),
      tm_w1, tm_b1.reshape(64, 1), tm_w2, tm_b2.reshape(32, 1),
      tm_w3, tm_b3.reshape(1, 1))
    return out_t.reshape(_B, 1)
